# Initial kernel scaffold; baseline (speedup 1.0000x reference)
import jax
import jax.numpy as jnp
from jax.experimental import pallas as pl


def kernel(x):
    raise NotImplementedError("write your pallas kernel here")


# speedup vs baseline: 1.5475x; 1.5475x over previous
import functools

import numpy as np
import jax
import jax.numpy as jnp
from jax import lax
from jax.experimental import pallas as pl
from jax.experimental.pallas import tpu as pltpu
from jax.experimental.pallas import tpu_sc as plsc

_N, _T, _D = 128, 2048, 128


def _crop_constants():
    np.random.seed(1)
    crop_l = int(np.random.randint(low=2, high=_T + 1))
    crop_left = int(np.random.randint(_T - crop_l + 1))
    crop_right = crop_left + crop_l
    crop_eleft = int(np.random.randint(crop_left + 1))
    crop_eright = int(np.random.randint(low=crop_right, high=_T + 1))
    crop_offset = np.random.randint(
        low=-crop_eleft, high=_T - crop_eright + 1, size=_N
    )
    return crop_l, crop_left, crop_eleft, crop_offset


_CROP_L, _CROP_LEFT, _CROP_ELEFT, _OFF = _crop_constants()
_START = _OFF + _CROP_LEFT
_B = _N * _CROP_L
_ROW_IDX = (
    np.arange(_N, dtype=np.int64)[:, None] * _T
    + _START[:, None]
    + np.arange(_CROP_L, dtype=np.int64)[None, :]
).reshape(-1).astype(np.int32)

_C = 128
_NCH = _B // _C
assert _NCH * _C == _B

_NC, _NS = 2, 16
_NW = _NC * _NS


def _gather_rows(x2d, idx):
    mesh = plsc.VectorSubcoreMesh(core_axis_name="c", subcore_axis_name="s")

    @functools.partial(
        pl.kernel,
        mesh=mesh,
        out_type=jax.ShapeDtypeStruct((_B, _D), jnp.float32),
        scratch_types=[
            pltpu.VMEM((_C,), jnp.int32),
            pltpu.VMEM((_C, _D), jnp.float32),
            pltpu.SemaphoreType.DMA,
        ],
    )
    def k(x_hbm, idx_hbm, out_hbm, idx_v, rows_v, sem):
        wid = lax.axis_index("s") * _NC + lax.axis_index("c")
        nmine = (_NCH - wid + _NW - 1) // _NW

        def step(i, carry):
            g = wid + i * _NW
            base = pl.multiple_of(g * _C, _C)
            pltpu.sync_copy(idx_hbm.at[pl.ds(base, _C)], idx_v)
            pltpu.async_copy(x_hbm.at[idx_v], rows_v, sem).wait()
            pltpu.sync_copy(rows_v, out_hbm.at[pl.ds(base, _C)])
            return carry

        lax.fori_loop(0, nmine, step, 0)

    return k(x2d, idx)


def kernel(x):
    x2d = x.reshape(_N * _T, _D)
    idx = jnp.asarray(_ROW_IDX)
    out = _gather_rows(x2d, idx)
    s = out.reshape(_N, _CROP_L, _D)
    left1 = jnp.asarray(_OFF + _CROP_ELEFT, dtype=jnp.int32)
    left2 = jnp.asarray(_START, dtype=jnp.int32)
    return (s, left1, s, left2, jnp.asarray(_CROP_L))


# speedup vs baseline: 1.6951x; 1.0954x over previous
import functools

import numpy as np
import jax
import jax.numpy as jnp
from jax import lax
from jax.experimental import pallas as pl
from jax.experimental.pallas import tpu as pltpu
from jax.experimental.pallas import tpu_sc as plsc

_N, _T, _D = 128, 2048, 128


def _crop_constants():
    np.random.seed(1)
    crop_l = int(np.random.randint(low=2, high=_T + 1))
    crop_left = int(np.random.randint(_T - crop_l + 1))
    crop_right = crop_left + crop_l
    crop_eleft = int(np.random.randint(crop_left + 1))
    crop_eright = int(np.random.randint(low=crop_right, high=_T + 1))
    crop_offset = np.random.randint(
        low=-crop_eleft, high=_T - crop_eright + 1, size=_N
    )
    return crop_l, crop_left, crop_eleft, crop_offset


_CROP_L, _CROP_LEFT, _CROP_ELEFT, _OFF = _crop_constants()
_START = _OFF + _CROP_LEFT
_B = _N * _CROP_L
_ROW_IDX = (
    np.arange(_N, dtype=np.int64)[:, None] * _T
    + _START[:, None]
    + np.arange(_CROP_L, dtype=np.int64)[None, :]
).reshape(-1).astype(np.int32)

_C = 128
_NCH = _B // _C
assert _NCH * _C == _B

_NC, _NS = 2, 16
_NW = _NC * _NS
_NI = -(-_NCH // _NW)


def _worker_idx():
    idx3 = np.zeros((_NW, _NI, _C), np.int32)
    for w in range(_NW):
        gs = list(range(w, _NCH, _NW))
        for i, g in enumerate(gs):
            idx3[w, i] = _ROW_IDX[g * _C:(g + 1) * _C]
    return idx3


_IDX3 = _worker_idx()


def _gather_rows(x2d, idx3):
    mesh = plsc.VectorSubcoreMesh(core_axis_name="c", subcore_axis_name="s")

    @functools.partial(
        pl.kernel,
        mesh=mesh,
        out_type=jax.ShapeDtypeStruct((_B, _D), jnp.float32),
        scratch_types=[
            pltpu.VMEM((_NI, _C), jnp.int32),
            pltpu.VMEM((_C, _D), jnp.float32),
            pltpu.VMEM((_C, _D), jnp.float32),
            pltpu.SemaphoreType.DMA,
            pltpu.SemaphoreType.DMA,
        ],
    )
    def k(x_hbm, idx_hbm, out_hbm, idx_v, rows0, rows1, sem0, sem1):
        wid = lax.axis_index("s") * _NC + lax.axis_index("c")
        last = (_NCH - 1 - wid) // _NW
        pltpu.sync_copy(idx_hbm.at[wid], idx_v)
        rows = (rows0, rows1)
        sems = (sem0, sem1)

        def src(i):
            ii = jnp.minimum(i, last)
            return x_hbm.at[idx_v.at[ii]]

        def dst(i):
            ii = jnp.minimum(i, last)
            base = pl.multiple_of((wid + ii * _NW) * _C, _C)
            return out_hbm.at[pl.ds(base, _C)]

        pltpu.make_async_copy(src(0), rows0, sem0).start()
        for i in range(_NI):
            b = i % 2
            pltpu.make_async_copy(src(i), rows[b], sems[b]).wait()
            if i + 1 < _NI:
                pltpu.make_async_copy(
                    src(i + 1), rows[1 - b], sems[1 - b]
                ).start()
            pltpu.sync_copy(rows[b], dst(i))

    return k(x2d, idx3)


def kernel(x):
    x2d = x.reshape(_N * _T, _D)
    idx3 = jnp.asarray(_IDX3)
    out = _gather_rows(x2d, idx3)
    s = out.reshape(_N, _CROP_L, _D)
    left1 = jnp.asarray(_OFF + _CROP_ELEFT, dtype=jnp.int32)
    left2 = jnp.asarray(_START, dtype=jnp.int32)
    return (s, left1, s, left2, jnp.asarray(_CROP_L))


# speedup vs baseline: 1.8148x; 1.0706x over previous
import functools

import numpy as np
import jax
import jax.numpy as jnp
from jax import lax
from jax.experimental import pallas as pl
from jax.experimental.pallas import tpu as pltpu
from jax.experimental.pallas import tpu_sc as plsc

_N, _T, _D = 128, 2048, 128


def _crop_constants():
    np.random.seed(1)
    crop_l = int(np.random.randint(low=2, high=_T + 1))
    crop_left = int(np.random.randint(_T - crop_l + 1))
    crop_right = crop_left + crop_l
    crop_eleft = int(np.random.randint(crop_left + 1))
    crop_eright = int(np.random.randint(low=crop_right, high=_T + 1))
    crop_offset = np.random.randint(
        low=-crop_eleft, high=_T - crop_eright + 1, size=_N
    )
    return crop_l, crop_left, crop_eleft, crop_offset


_CROP_L, _CROP_LEFT, _CROP_ELEFT, _OFF = _crop_constants()
_START = _OFF + _CROP_LEFT
_B = _N * _CROP_L

_NC, _NS = 2, 16
_NW = _NC * _NS
_RPW = _N // _NW
_CB = 448

_BASES = np.zeros((_NW, 16), np.int32)
for _w in range(_NW):
    for _j in range(_RPW):
        _r = _w * _RPW + _j
        _BASES[_w, _j] = _r * _T + _START[_r]

_CHUNKS = []
for _j in range(_RPW):
    _o = 0
    while _o < _CROP_L:
        _ln = min(_CB, _CROP_L - _o)
        _CHUNKS.append((_j, _o, _ln))
        _o += _ln


def _crop_copy(x1d):
    mesh = plsc.VectorSubcoreMesh(core_axis_name="c", subcore_axis_name="s")

    @functools.partial(
        pl.kernel,
        mesh=mesh,
        out_type=jax.ShapeDtypeStruct((_B * _D,), jnp.float32),
        scratch_types=[
            pltpu.VMEM((_CB * _D,), jnp.float32),
            pltpu.VMEM((_CB * _D,), jnp.float32),
            pltpu.SemaphoreType.DMA,
            pltpu.SemaphoreType.DMA,
        ],
    )
    def k(x_hbm, out_hbm, buf0, buf1, sem0, sem1):
        wid = lax.axis_index("s") * _NC + lax.axis_index("c")
        src_base = []
        for j in range(_RPW):
            b = jnp.int32(int(_BASES[0, j]))
            for w in range(1, _NW):
                b = jnp.where(wid == w, jnp.int32(int(_BASES[w, j])), b)
            src_base.append(b)
        bufs = (buf0, buf1)
        sems = (sem0, sem1)

        def src(t):
            j, o, ln = _CHUNKS[t]
            off = pl.multiple_of((src_base[j] + o) * _D, _D)
            return x_hbm.at[pl.ds(off, ln * _D)]

        def stage(t):
            j, o, ln = _CHUNKS[t]
            return bufs[t % 2].at[pl.ds(0, ln * _D)]

        def dst(t):
            j, o, ln = _CHUNKS[t]
            off = pl.multiple_of(((wid * _RPW + j) * _CROP_L + o) * _D, _D)
            return out_hbm.at[pl.ds(off, ln * _D)]

        pltpu.make_async_copy(src(0), stage(0), sems[0]).start()
        for t in range(len(_CHUNKS)):
            b = t % 2
            pltpu.make_async_copy(src(t), stage(t), sems[b]).wait()
            if t + 1 < len(_CHUNKS):
                pltpu.make_async_copy(
                    src(t + 1), stage(t + 1), sems[1 - b]
                ).start()
            pltpu.sync_copy(stage(t), dst(t))

    return k(x1d)


def kernel(x):
    x1d = x.reshape(_N * _T * _D)
    out = _crop_copy(x1d)
    s = out.reshape(_N, _CROP_L, _D)
    left1 = jnp.asarray(_OFF + _CROP_ELEFT, dtype=jnp.int32)
    left2 = jnp.asarray(_START, dtype=jnp.int32)
    return (s, left1, s, left2, jnp.asarray(_CROP_L))


# speedup vs baseline: 4.5246x; 2.4932x over previous
import functools

import numpy as np
import jax
import jax.numpy as jnp
from jax import lax
from jax.experimental import pallas as pl
from jax.experimental.pallas import tpu as pltpu
from jax.experimental.pallas import tpu_sc as plsc

_N, _T, _D = 128, 2048, 128


def _crop_constants():
    np.random.seed(1)
    crop_l = int(np.random.randint(low=2, high=_T + 1))
    crop_left = int(np.random.randint(_T - crop_l + 1))
    crop_right = crop_left + crop_l
    crop_eleft = int(np.random.randint(crop_left + 1))
    crop_eright = int(np.random.randint(low=crop_right, high=_T + 1))
    crop_offset = np.random.randint(
        low=-crop_eleft, high=_T - crop_eright + 1, size=_N
    )
    return crop_l, crop_left, crop_eleft, crop_offset


_CROP_L, _CROP_LEFT, _CROP_ELEFT, _OFF = _crop_constants()
_START = _OFF + _CROP_LEFT
_B = _N * _CROP_L

_NC, _NS = 2, 16
_NW = _NC * _NS
_RPW = _N // _NW
_CB = 448

_BASES = np.zeros((_NW, _RPW), np.int64)
for _w in range(_NW):
    for _j in range(_RPW):
        _r = _w * _RPW + _j
        _BASES[_w, _j] = _r * _T + _START[_r]

_CHUNKS = []
for _j in range(_RPW):
    _o = 0
    while _o < _CROP_L:
        _ln = min(_CB, _CROP_L - _o)
        if _ln % 8:
            _ln8 = -(-_ln // 8) * 8
            _CHUNKS.append((_j, _CROP_L - _ln8, _ln8))
            break
        _CHUNKS.append((_j, _o, _ln))
        _o += _ln


def _crop_copy(x2d):
    mesh = plsc.VectorSubcoreMesh(core_axis_name="c", subcore_axis_name="s")
    out_sds = jax.ShapeDtypeStruct((_CROP_L, _N, _D), jnp.float32)

    @functools.partial(
        pl.kernel,
        mesh=mesh,
        out_type=(out_sds, out_sds),
        compiler_params=pltpu.CompilerParams(use_tc_tiling_on_sc=False),
        scratch_types=[
            pltpu.VMEM((_CB, _D), jnp.float32),
            pltpu.VMEM((_CB, _D), jnp.float32),
            pltpu.SemaphoreType.DMA,
            pltpu.SemaphoreType.DMA,
        ],
    )
    def k(x_hbm, out_a, out_b, buf0, buf1, sem0, sem1):
        wid = lax.axis_index("s") * _NC + lax.axis_index("c")
        src_base = []
        for j in range(_RPW):
            b = jnp.int32(int(_BASES[0, j]))
            for w in range(1, _NW):
                b = jnp.where(wid == w, jnp.int32(int(_BASES[w, j])), b)
            src_base.append(b)
        bufs = (buf0, buf1)
        sems = (sem0, sem1)

        def src(t):
            j, o, ln = _CHUNKS[t]
            off = src_base[j] + o
            return x_hbm.at[pl.ds(off, ln)]

        def stage(t):
            j, o, ln = _CHUNKS[t]
            return bufs[t % 2].at[pl.ds(0, ln)]

        def dst(t, out_hbm):
            j, o, ln = _CHUNKS[t]
            rb = wid * _RPW + j
            return out_hbm.at[pl.ds(o, ln), rb]

        pltpu.make_async_copy(src(0), stage(0), sems[0]).start()
        for t in range(len(_CHUNKS)):
            b = t % 2
            pltpu.make_async_copy(src(t), stage(t), sems[b]).wait()
            if t + 1 < len(_CHUNKS):
                pltpu.make_async_copy(
                    src(t + 1), stage(t + 1), sems[1 - b]
                ).start()
            pltpu.sync_copy(stage(t), dst(t, out_a))
            pltpu.sync_copy(stage(t), dst(t, out_b))

    return k(x2d)


def kernel(x):
    x2d = x.reshape(_N * _T, _D)
    out_a, out_b = _crop_copy(x2d)
    s1 = jnp.transpose(out_a, (1, 0, 2))
    s2 = jnp.transpose(out_b, (1, 0, 2))
    left1 = jnp.asarray(_OFF + _CROP_ELEFT, dtype=jnp.int32)
    left2 = jnp.asarray(_START, dtype=jnp.int32)
    return (s1, left1, s2, left2, jnp.asarray(_CROP_L))


# speedup vs baseline: 4.5404x; 1.0035x over previous
import functools

import numpy as np
import jax
import jax.numpy as jnp
from jax import lax
from jax.experimental import pallas as pl
from jax.experimental.pallas import tpu as pltpu
from jax.experimental.pallas import tpu_sc as plsc

_N, _T, _D = 128, 2048, 128


def _crop_constants():
    np.random.seed(1)
    crop_l = int(np.random.randint(low=2, high=_T + 1))
    crop_left = int(np.random.randint(_T - crop_l + 1))
    crop_right = crop_left + crop_l
    crop_eleft = int(np.random.randint(crop_left + 1))
    crop_eright = int(np.random.randint(low=crop_right, high=_T + 1))
    crop_offset = np.random.randint(
        low=-crop_eleft, high=_T - crop_eright + 1, size=_N
    )
    return crop_l, crop_left, crop_eleft, crop_offset


_CROP_L, _CROP_LEFT, _CROP_ELEFT, _OFF = _crop_constants()
_START = _OFF + _CROP_LEFT
_B = _N * _CROP_L

_NC, _NS = 2, 16
_NW = _NC * _NS
_RPW = _N // _NW
_CB = 448

_BASES = np.zeros((_NW, _RPW), np.int64)
for _w in range(_NW):
    for _j in range(_RPW):
        _r = _w * _RPW + _j
        _BASES[_w, _j] = _r * _T + _START[_r]

_CHUNKS = []
for _j in range(_RPW):
    _o = 0
    while _o < _CROP_L:
        _ln = min(_CB, _CROP_L - _o)
        if _ln % 8:
            _ln8 = -(-_ln // 8) * 8
            _CHUNKS.append((_j, _CROP_L - _ln8, _ln8))
            break
        _CHUNKS.append((_j, _o, _ln))
        _o += _ln


def _crop_copy(x2d):
    mesh = plsc.VectorSubcoreMesh(core_axis_name="c", subcore_axis_name="s")
    out_sds = jax.ShapeDtypeStruct((_CROP_L, _N, _D), jnp.float32)

    @functools.partial(
        pl.kernel,
        mesh=mesh,
        out_type=(out_sds, out_sds),
        compiler_params=pltpu.CompilerParams(use_tc_tiling_on_sc=False),
        scratch_types=[
            pltpu.VMEM((_CB, _D), jnp.float32),
            pltpu.VMEM((_CB, _D), jnp.float32),
            pltpu.SemaphoreType.DMA,
            pltpu.SemaphoreType.DMA,
            pltpu.SemaphoreType.DMA,
            pltpu.SemaphoreType.DMA,
            pltpu.SemaphoreType.DMA,
            pltpu.SemaphoreType.DMA,
        ],
    )
    def k(x_hbm, out_a, out_b, buf0, buf1, sem0, sem1, wa0, wa1, wb0, wb1):
        wid = lax.axis_index("s") * _NC + lax.axis_index("c")
        src_base = []
        for j in range(_RPW):
            b = jnp.int32(int(_BASES[0, j]))
            for w in range(1, _NW):
                b = jnp.where(wid == w, jnp.int32(int(_BASES[w, j])), b)
            src_base.append(b)
        bufs = (buf0, buf1)
        sems = (sem0, sem1)
        wsems_a = (wa0, wa1)
        wsems_b = (wb0, wb1)

        def src(t):
            j, o, ln = _CHUNKS[t]
            off = src_base[j] + o
            return x_hbm.at[pl.ds(off, ln)]

        def stage(t):
            j, o, ln = _CHUNKS[t]
            return bufs[t % 2].at[pl.ds(0, ln)]

        def dst(t, out_hbm):
            j, o, ln = _CHUNKS[t]
            rb = wid * _RPW + j
            return out_hbm.at[pl.ds(o, ln), rb]

        def wcopy_a(t):
            return pltpu.make_async_copy(
                stage(t), dst(t, out_a), wsems_a[t % 2]
            )

        def wcopy_b(t):
            return pltpu.make_async_copy(
                stage(t), dst(t, out_b), wsems_b[t % 2]
            )

        n = len(_CHUNKS)
        pltpu.make_async_copy(src(0), stage(0), sems[0]).start()
        for t in range(n):
            b = t % 2
            pltpu.make_async_copy(src(t), stage(t), sems[b]).wait()
            if t + 1 < n:
                if t >= 1:
                    wcopy_a(t - 1).wait()
                    wcopy_b(t - 1).wait()
                pltpu.make_async_copy(
                    src(t + 1), stage(t + 1), sems[1 - b]
                ).start()
            wcopy_a(t).start()
            wcopy_b(t).start()
        if n >= 2:
            wcopy_a(n - 2).wait()
            wcopy_b(n - 2).wait()
        wcopy_a(n - 1).wait()
        wcopy_b(n - 1).wait()

    return k(x2d)


def kernel(x):
    x2d = x.reshape(_N * _T, _D)
    out_a, out_b = _crop_copy(x2d)
    s1 = jnp.transpose(out_a, (1, 0, 2))
    s2 = jnp.transpose(out_b, (1, 0, 2))
    left1 = jnp.asarray(_OFF + _CROP_ELEFT, dtype=jnp.int32)
    left2 = jnp.asarray(_START, dtype=jnp.int32)
    return (s1, left1, s2, left2, jnp.asarray(_CROP_L))
